# initial kernel scaffold (unmeasured)
import os

import jax
import jax.numpy as jnp
from jax import lax
from jax.experimental import pallas as pl
from jax.experimental.pallas import tpu as pltpu

N_DEV = 8
HP = 8
DH = 128
SQ = 2048
SKV = 2048
DM = 1024
NCH = 8
CH = SQ // NCH
QT = 512
NQT = SQ // NQT if False else SQ // QT
SCALE = 0.08838834764831843
N_HOPS = 2 * (N_DEV - 1)

_INTERPRET = os.environ.get("SCBAND_INTERPRET") == "1"


def kernel(x, Wq, K_ext, V_ext, Wo):
    def body(x_ref, wq_ref, k_hbm, v_hbm, wo_ref, out_ref,
             k_vmem, v_vmem, q_ref, ctx_ref, send_buf, recv_buf,
             kv_sems, send_sems, recv_sems):
        my = lax.axis_index("i")
        left = lax.rem(my + N_DEV - 1, N_DEV)
        right = lax.rem(my + 1, N_DEV)

        barrier = pltpu.get_barrier_semaphore()
        for nbr in (left, right):
            pl.semaphore_signal(barrier, inc=1, device_id=(nbr,),
                                device_id_type=pl.DeviceIdType.MESH)
        pl.semaphore_wait(barrier, 2)

        copies = []
        for j in range(HP):
            h = my * HP + j
            ck = pltpu.make_async_copy(
                k_hbm.at[0, :, h, :], k_vmem.at[j], kv_sems.at[j])
            cv = pltpu.make_async_copy(
                v_hbm.at[0, :, h, :], v_vmem.at[j], kv_sems.at[HP + j])
            ck.start()
            cv.start()
            copies.append(ck)
            copies.append(cv)

        xb = x_ref[0].astype(jnp.bfloat16)
        wqb = wq_ref[...].astype(jnp.bfloat16)
        qfull = lax.dot_general(
            xb, wqb, (((1,), (0,)), ((), ())),
            preferred_element_type=jnp.float32).astype(jnp.bfloat16)
        for j in range(HP):
            q_ref[j] = qfull[:, j * DH:(j + 1) * DH]

        for c in copies:
            c.wait()

        def attn_step(i, carry):
            h = i // NQT
            t = i % NQT
            qh = q_ref[h, pl.ds(t * QT, QT), :]
            kh = k_vmem[h].astype(jnp.bfloat16)
            vh = v_vmem[h].astype(jnp.bfloat16)
            s = lax.dot_general(
                qh, kh, (((1,), (1,)), ((), ())),
                preferred_element_type=jnp.float32) * SCALE
            qi = lax.broadcasted_iota(jnp.int32, (QT, SKV), 0) + t * QT
            ki = lax.broadcasted_iota(jnp.int32, (QT, SKV), 1)
            mask = (jnp.abs(qi - ki) <= 128) | (ki < 32) | (qi < 32)
            s = jnp.where(mask, s, -1e9)
            m = jnp.max(s, axis=1, keepdims=True)
            w = jnp.exp(s - m)
            denom = jnp.sum(w, axis=1, keepdims=True)
            p = (w / denom).astype(jnp.bfloat16)
            ctx = lax.dot_general(
                p, vh, (((1,), (0,)), ((), ())),
                preferred_element_type=jnp.float32)
            ctx_ref[h, pl.ds(t * QT, QT), :] = ctx.astype(jnp.bfloat16)
            return carry

        lax.fori_loop(0, HP * NQT, attn_step, 0)

        acc = jnp.zeros((SQ, DM), jnp.float32)
        for j in range(HP):
            wob = wo_ref[j * DH:(j + 1) * DH, :].astype(jnp.bfloat16)
            acc = acc + lax.dot_general(
                ctx_ref[j], wob, (((1,), (0,)), ((), ())),
                preferred_element_type=jnp.float32)
        out_ref[0] = acc

        def hop(slot, chunk, store_fn):
            off = chunk * CH
            send_buf[...] = out_ref[0, pl.ds(off, CH), :]
            rdma = pltpu.make_async_remote_copy(
                src_ref=send_buf,
                dst_ref=recv_buf.at[slot],
                send_sem=send_sems.at[slot],
                recv_sem=recv_sems.at[slot],
                device_id=(right,),
                device_id_type=pl.DeviceIdType.MESH)
            rdma.start()
            rdma.wait()
            store_fn()

        for s_ in range(N_DEV - 1):
            chunk = lax.rem(my - s_ + N_DEV, N_DEV)
            rchunk = lax.rem(my - s_ - 1 + N_DEV, N_DEV)

            def store(rc=rchunk, slot=s_):
                off = rc * CH
                out_ref[0, pl.ds(off, CH), :] = (
                    out_ref[0, pl.ds(off, CH), :] + recv_buf[slot])

            hop(s_, chunk, store)

        for s_ in range(N_DEV - 1):
            chunk = lax.rem(my + 1 - s_ + N_DEV, N_DEV)
            rchunk = lax.rem(my - s_ + N_DEV, N_DEV)
            slot = N_DEV - 1 + s_

            def store(rc=rchunk, sl=slot):
                out_ref[0, pl.ds(rc * CH, CH), :] = recv_buf[sl]

            hop(slot, chunk, store)

    kwargs = {}
    if _INTERPRET:
        kwargs["interpret"] = pltpu.InterpretParams()

    return pl.pallas_call(
        body,
        out_shape=jax.ShapeDtypeStruct((1, SQ, DM), jnp.float32),
        in_specs=[
            pl.BlockSpec(memory_space=pltpu.MemorySpace.VMEM),
            pl.BlockSpec(memory_space=pltpu.MemorySpace.VMEM),
            pl.BlockSpec(memory_space=pltpu.MemorySpace.HBM),
            pl.BlockSpec(memory_space=pltpu.MemorySpace.HBM),
            pl.BlockSpec(memory_space=pltpu.MemorySpace.VMEM),
        ],
        out_specs=pl.BlockSpec(memory_space=pltpu.MemorySpace.VMEM),
        scratch_shapes=[
            pltpu.VMEM((HP, SKV, DH), jnp.float32),
            pltpu.VMEM((HP, SKV, DH), jnp.float32),
            pltpu.VMEM((HP, SQ, DH), jnp.bfloat16),
            pltpu.VMEM((HP, SQ, DH), jnp.bfloat16),
            pltpu.VMEM((CH, DM), jnp.float32),
            pltpu.VMEM((N_HOPS, CH, DM), jnp.float32),
            pltpu.SemaphoreType.DMA((2 * HP,)),
            pltpu.SemaphoreType.DMA((N_HOPS,)),
            pltpu.SemaphoreType.DMA((N_HOPS,)),
        ],
        compiler_params=pltpu.CompilerParams(
            collective_id=0,
            vmem_limit_bytes=110 * 1024 * 1024,
        ),
    )(x, Wq, K_ext, V_ext, Wo)


# baseline (device time: 414811 ns/iter reference)
import os

import jax
import jax.numpy as jnp
from jax import lax
from jax.experimental import pallas as pl
from jax.experimental.pallas import tpu as pltpu

N_DEV = 8
HP = 8
DH = 128
SQ = 2048
SKV = 2048
DM = 1024
CH = SQ // N_DEV
QT = 512
NQT = SQ // QT
SCALE = 0.08838834764831843
N_HOPS = 2 * (N_DEV - 1)

_INTERPRET = os.environ.get("SCBAND_INTERPRET") == "1"


def kernel(x, Wq, K_ext, V_ext, Wo):
    xb = x.astype(jnp.bfloat16)
    wqb = Wq.astype(jnp.bfloat16)
    wob = Wo.astype(jnp.bfloat16)
    kb = jnp.transpose(K_ext[0].astype(jnp.bfloat16), (1, 0, 2))
    vb = jnp.transpose(V_ext[0].astype(jnp.bfloat16), (1, 0, 2))

    def body(x_ref, wq_ref, k_hbm, v_hbm, wo_ref, out_ref,
             k_vmem, v_vmem, q_ref, ctx_ref, recv_buf,
             kv_sems, send_sems, recv_sems):
        my = lax.axis_index("i")
        left = lax.rem(my + N_DEV - 1, N_DEV)
        right = lax.rem(my + 1, N_DEV)

        barrier = pltpu.get_barrier_semaphore()
        for nbr in (left, right):
            pl.semaphore_signal(barrier, inc=1, device_id=(nbr,),
                                device_id_type=pl.DeviceIdType.MESH)
        pl.semaphore_wait(barrier, 2)

        copies = [
            pltpu.make_async_copy(
                k_hbm.at[pl.ds(my * HP, HP)], k_vmem, kv_sems.at[0]),
            pltpu.make_async_copy(
                v_hbm.at[pl.ds(my * HP, HP)], v_vmem, kv_sems.at[1]),
        ]
        for c in copies:
            c.start()

        for rt in range(NQT):
            r0 = rt * QT
            qc = lax.dot_general(
                x_ref[0, r0:r0 + QT, :], wq_ref[...],
                (((1,), (0,)), ((), ())),
                preferred_element_type=jnp.float32).astype(jnp.bfloat16)
            for j in range(HP):
                q_ref[j, r0:r0 + QT, :] = qc[:, j * DH:(j + 1) * DH]

        for c in copies:
            c.wait()

        def attn_step(i, carry):
            h = i // NQT
            t = i % NQT
            qh = q_ref[h, pl.ds(t * QT, QT), :]
            s = lax.dot_general(
                qh, k_vmem[h], (((1,), (1,)), ((), ())),
                preferred_element_type=jnp.float32) * SCALE
            qi = lax.broadcasted_iota(jnp.int32, (QT, SKV), 0) + t * QT
            ki = lax.broadcasted_iota(jnp.int32, (QT, SKV), 1)
            mask = (jnp.abs(qi - ki) <= 128) | (ki < 32) | (qi < 32)
            s = jnp.where(mask, s, -1e9)
            m = jnp.max(s, axis=1, keepdims=True)
            w = jnp.exp(s - m)
            denom = jnp.sum(w, axis=1, keepdims=True)
            p = (w / denom).astype(jnp.bfloat16)
            ctx = lax.dot_general(
                p, v_vmem[h], (((1,), (0,)), ((), ())),
                preferred_element_type=jnp.float32)
            ctx_ref[h, pl.ds(t * QT, QT), :] = ctx.astype(jnp.bfloat16)
            return carry

        lax.fori_loop(0, HP * NQT, attn_step, 0)

        for rt in range(NQT):
            r0 = rt * QT
            acc = jnp.zeros((QT, DM), jnp.float32)
            for j in range(HP):
                acc = acc + lax.dot_general(
                    ctx_ref[j, r0:r0 + QT, :], wo_ref[j * DH:(j + 1) * DH, :],
                    (((1,), (0,)), ((), ())),
                    preferred_element_type=jnp.float32)
            out_ref[0, r0:r0 + QT, :] = acc

        def send_chunk(slot, chunk, dst_ref):
            rdma = pltpu.make_async_remote_copy(
                src_ref=out_ref.at[0, pl.ds(chunk * CH, CH), :],
                dst_ref=dst_ref,
                send_sem=send_sems.at[slot],
                recv_sem=recv_sems.at[slot],
                device_id=(right,),
                device_id_type=pl.DeviceIdType.MESH)
            rdma.start()
            rdma.wait()

        for s_ in range(N_DEV - 1):
            chunk = lax.rem(my - s_ + N_DEV, N_DEV)
            send_chunk(s_, chunk, recv_buf.at[s_])
            roff = lax.rem(my - s_ - 1 + N_DEV, N_DEV) * CH
            out_ref[0, pl.ds(roff, CH), :] = (
                out_ref[0, pl.ds(roff, CH), :] + recv_buf[s_])

        for s_ in range(N_DEV - 1):
            chunk = lax.rem(my + 1 - s_ + N_DEV, N_DEV)
            send_chunk(N_DEV - 1 + s_, chunk,
                       out_ref.at[0, pl.ds(chunk * CH, CH), :])

    kwargs = {}
    if _INTERPRET:
        kwargs["interpret"] = pltpu.InterpretParams()

    return pl.pallas_call(
        body,
        out_shape=jax.ShapeDtypeStruct((1, SQ, DM), jnp.float32),
        in_specs=[
            pl.BlockSpec(memory_space=pltpu.MemorySpace.VMEM),
            pl.BlockSpec(memory_space=pltpu.MemorySpace.VMEM),
            pl.BlockSpec(memory_space=pltpu.MemorySpace.HBM),
            pl.BlockSpec(memory_space=pltpu.MemorySpace.HBM),
            pl.BlockSpec(memory_space=pltpu.MemorySpace.VMEM),
        ],
        out_specs=pl.BlockSpec(memory_space=pltpu.MemorySpace.VMEM),
        scratch_shapes=[
            pltpu.VMEM((HP, SKV, DH), jnp.bfloat16),
            pltpu.VMEM((HP, SKV, DH), jnp.bfloat16),
            pltpu.VMEM((HP, SQ, DH), jnp.bfloat16),
            pltpu.VMEM((HP, SQ, DH), jnp.bfloat16),
            pltpu.VMEM((N_DEV - 1, CH, DM), jnp.float32),
            pltpu.SemaphoreType.DMA((2,)),
            pltpu.SemaphoreType.DMA((N_HOPS,)),
            pltpu.SemaphoreType.DMA((N_HOPS,)),
        ],
        compiler_params=pltpu.CompilerParams(
            collective_id=0,
            vmem_limit_bytes=60 * 1024 * 1024,
        ),
        **kwargs,
    )(xb, wqb, kb, vb, wob)


# device time: 240656 ns/iter; 1.7237x vs baseline; 1.7237x over previous
import os

import jax
import jax.numpy as jnp
from jax import lax
from jax.experimental import pallas as pl
from jax.experimental.pallas import tpu as pltpu

N_DEV = 8
HP = 8
DH = 128
SQ = 2048
SKV = 2048
DM = 1024
HC = DM // 2
CH = SQ // N_DEV
QT = 512
NQT = SQ // QT
SCALE = 0.08838834764831843
N_HOPS = 2 * (N_DEV - 1)

_INTERPRET = os.environ.get("SCBAND_INTERPRET") == "1"


def kernel(x, Wq, K_ext, V_ext, Wo):
    my_out = lax.axis_index("i")
    xb = x.astype(jnp.bfloat16)
    wqb = Wq.astype(jnp.bfloat16)
    wob = Wo.astype(jnp.bfloat16)
    kb = jnp.transpose(
        lax.dynamic_slice(K_ext[0], (0, my_out * HP, 0), (SKV, HP, DH)),
        (1, 0, 2)).astype(jnp.bfloat16)
    vb = jnp.transpose(
        lax.dynamic_slice(V_ext[0], (0, my_out * HP, 0), (SKV, HP, DH)),
        (1, 0, 2)).astype(jnp.bfloat16)

    def body(x_ref, wq_ref, k_ref, v_ref, wo_ref, out_ref,
             q_ref, ctx_ref, agb, rsb_p, rsb_m, ssb_p, ssb_m,
             send_p, recv_p, send_m, recv_m):
        my = lax.axis_index("i")
        left = lax.rem(my + N_DEV - 1, N_DEV)
        right = lax.rem(my + 1, N_DEV)

        barrier = pltpu.get_barrier_semaphore()
        for nbr in (left, right):
            pl.semaphore_signal(barrier, inc=1, device_id=(nbr,),
                                device_id_type=pl.DeviceIdType.MESH)
        pl.semaphore_wait(barrier, 2)

        for rt in range(NQT):
            r0 = rt * QT
            qc = lax.dot_general(
                x_ref[0, r0:r0 + QT, :], wq_ref[...],
                (((1,), (0,)), ((), ())),
                preferred_element_type=jnp.float32).astype(jnp.bfloat16)
            for j in range(HP):
                q_ref[j, r0:r0 + QT, :] = qc[:, j * DH:(j + 1) * DH]

        def attn_step(i, carry):
            h = i // NQT
            t = i % NQT
            qh = q_ref[h, pl.ds(t * QT, QT), :]
            s = lax.dot_general(
                qh, k_ref[h], (((1,), (1,)), ((), ())),
                preferred_element_type=jnp.float32) * SCALE
            qi = lax.broadcasted_iota(jnp.int32, (QT, SKV), 0) + t * QT
            ki = lax.broadcasted_iota(jnp.int32, (QT, SKV), 1)
            mask = (jnp.abs(qi - ki) <= 128) | (ki < 32) | (qi < 32)
            s = jnp.where(mask, s, -1e9)
            m = jnp.max(s, axis=1, keepdims=True)
            w = jnp.exp(s - m)
            denom = jnp.sum(w, axis=1, keepdims=True)
            p = (w / denom).astype(jnp.bfloat16)
            ctx = lax.dot_general(
                p, v_ref[h], (((1,), (0,)), ((), ())),
                preferred_element_type=jnp.float32)
            ctx_ref[h, pl.ds(t * QT, QT), :] = ctx.astype(jnp.bfloat16)
            return carry

        lax.fori_loop(0, HP * NQT, attn_step, 0)

        for rt in range(NQT):
            r0 = rt * QT
            acc = jnp.zeros((QT, DM), jnp.float32)
            for j in range(HP):
                acc = acc + lax.dot_general(
                    ctx_ref[j, r0:r0 + QT, :], wo_ref[j * DH:(j + 1) * DH, :],
                    (((1,), (0,)), ((), ())),
                    preferred_element_type=jnp.float32)
            out_ref[0, r0:r0 + QT, :] = acc

        def rdma(slot, src, dst, dev, sems):
            r = pltpu.make_async_remote_copy(
                src_ref=src, dst_ref=dst,
                send_sem=sems[0].at[slot], recv_sem=sems[1].at[slot],
                device_id=(dev,), device_id_type=pl.DeviceIdType.MESH)
            r.start()
            return r

        for s_ in range(N_DEV - 1):
            cp = lax.rem(my - s_ + N_DEV, N_DEV)
            cm = lax.rem(my + s_, N_DEV)
            ssb_p[...] = out_ref[0, pl.ds(cp * CH, CH), 0:HC].astype(jnp.bfloat16)
            ssb_m[...] = out_ref[0, pl.ds(cm * CH, CH), HC:DM].astype(jnp.bfloat16)
            rp = rdma(s_, ssb_p, rsb_p.at[s_], right, (send_p, recv_p))
            rm = rdma(s_, ssb_m, rsb_m.at[s_], left, (send_m, recv_m))
            rp.wait()
            rm.wait()
            op = lax.rem(my - s_ - 1 + N_DEV, N_DEV) * CH
            om = lax.rem(my + s_ + 1, N_DEV) * CH
            out_ref[0, pl.ds(op, CH), 0:HC] = (
                out_ref[0, pl.ds(op, CH), 0:HC] + rsb_p[s_].astype(jnp.float32))
            out_ref[0, pl.ds(om, CH), HC:DM] = (
                out_ref[0, pl.ds(om, CH), HC:DM] + rsb_m[s_].astype(jnp.float32))

        o_p = lax.rem(my + 1, N_DEV) * CH
        o_m = lax.rem(my - 1 + N_DEV, N_DEV) * CH
        agb[pl.ds(o_p, CH), 0:HC] = out_ref[0, pl.ds(o_p, CH), 0:HC].astype(jnp.bfloat16)
        agb[pl.ds(o_m, CH), HC:DM] = out_ref[0, pl.ds(o_m, CH), HC:DM].astype(jnp.bfloat16)

        for s_ in range(N_DEV - 1):
            slot = N_DEV - 1 + s_
            sp = lax.rem(my + 1 - s_ + N_DEV, N_DEV) * CH
            sm = lax.rem(my - 1 + s_ + N_DEV, N_DEV) * CH
            rp = rdma(slot, agb.at[pl.ds(sp, CH), 0:HC],
                      agb.at[pl.ds(sp, CH), 0:HC], right, (send_p, recv_p))
            rm = rdma(slot, agb.at[pl.ds(sm, CH), HC:DM],
                      agb.at[pl.ds(sm, CH), HC:DM], left, (send_m, recv_m))
            rp.wait()
            rm.wait()
            gp = lax.rem(my - s_ + N_DEV, N_DEV) * CH
            gm = lax.rem(my + s_, N_DEV) * CH
            out_ref[0, pl.ds(gp, CH), 0:HC] = agb[pl.ds(gp, CH), 0:HC].astype(jnp.float32)
            out_ref[0, pl.ds(gm, CH), HC:DM] = agb[pl.ds(gm, CH), HC:DM].astype(jnp.float32)

    kwargs = {}
    if _INTERPRET:
        kwargs["interpret"] = pltpu.InterpretParams()

    return pl.pallas_call(
        body,
        out_shape=jax.ShapeDtypeStruct((1, SQ, DM), jnp.float32),
        in_specs=[
            pl.BlockSpec(memory_space=pltpu.MemorySpace.VMEM),
            pl.BlockSpec(memory_space=pltpu.MemorySpace.VMEM),
            pl.BlockSpec(memory_space=pltpu.MemorySpace.VMEM),
            pl.BlockSpec(memory_space=pltpu.MemorySpace.VMEM),
            pl.BlockSpec(memory_space=pltpu.MemorySpace.VMEM),
        ],
        out_specs=pl.BlockSpec(memory_space=pltpu.MemorySpace.VMEM),
        scratch_shapes=[
            pltpu.VMEM((HP, SQ, DH), jnp.bfloat16),
            pltpu.VMEM((HP, SQ, DH), jnp.bfloat16),
            pltpu.VMEM((SQ, DM), jnp.bfloat16),
            pltpu.VMEM((N_DEV - 1, CH, HC), jnp.bfloat16),
            pltpu.VMEM((N_DEV - 1, CH, HC), jnp.bfloat16),
            pltpu.VMEM((CH, HC), jnp.bfloat16),
            pltpu.VMEM((CH, HC), jnp.bfloat16),
            pltpu.SemaphoreType.DMA((N_HOPS,)),
            pltpu.SemaphoreType.DMA((N_HOPS,)),
            pltpu.SemaphoreType.DMA((N_HOPS,)),
            pltpu.SemaphoreType.DMA((N_HOPS,)),
        ],
        compiler_params=pltpu.CompilerParams(
            collective_id=0,
            vmem_limit_bytes=60 * 1024 * 1024,
        ),
        **kwargs,
    )(xb, wqb, kb, vb, wob)


# device time: 184503 ns/iter; 2.2483x vs baseline; 1.3043x over previous
import os

import jax
import jax.numpy as jnp
from jax import lax
from jax.experimental import pallas as pl
from jax.experimental.pallas import tpu as pltpu

N_DEV = 8
HP = 8
DH = 128
SQ = 2048
SKV = 2048
DM = 1024
HC = DM // 2
CH = SQ // N_DEV
QT = 512
NQT = SQ // QT
SCALE = 0.08838834764831843
N_HOPS = 2 * (N_DEV - 1)

_INTERPRET = os.environ.get("SCBAND_INTERPRET") == "1"


def kernel(x, Wq, K_ext, V_ext, Wo):
    my_out = lax.axis_index("i")
    xb = x.astype(jnp.bfloat16)
    wqb = Wq.astype(jnp.bfloat16)
    wob = Wo.astype(jnp.bfloat16)
    kb = jnp.transpose(
        lax.dynamic_slice(K_ext[0], (0, my_out * HP, 0), (SKV, HP, DH)),
        (1, 0, 2)).astype(jnp.bfloat16)
    vb = jnp.transpose(
        lax.dynamic_slice(V_ext[0], (0, my_out * HP, 0), (SKV, HP, DH)),
        (1, 0, 2)).astype(jnp.bfloat16)

    def body(x_ref, wq_ref, k_ref, v_ref, wo_ref, out_ref,
             q_ref, ctx_ref, agb, rsb_p, rsb_m, ssb_p, ssb_m,
             send_p, recv_p, send_m, recv_m):
        my = lax.axis_index("i")
        left = lax.rem(my + N_DEV - 1, N_DEV)
        right = lax.rem(my + 1, N_DEV)

        barrier = pltpu.get_barrier_semaphore()
        for nbr in (left, right):
            pl.semaphore_signal(barrier, inc=1, device_id=(nbr,),
                                device_id_type=pl.DeviceIdType.MESH)
        pl.semaphore_wait(barrier, 2)

        for rt in range(NQT):
            r0 = rt * QT
            qc = lax.dot_general(
                x_ref[0, r0:r0 + QT, :], wq_ref[...],
                (((1,), (0,)), ((), ())),
                preferred_element_type=jnp.float32).astype(jnp.bfloat16)
            for j in range(HP):
                q_ref[j, r0:r0 + QT, :] = qc[:, j * DH:(j + 1) * DH]

        def attn_t0(h, carry):
            qh = q_ref[h, 0:QT, :]
            s = lax.dot_general(
                qh, k_ref[h], (((1,), (1,)), ((), ())),
                preferred_element_type=jnp.float32) * SCALE
            qi = lax.broadcasted_iota(jnp.int32, (QT, SKV), 0)
            ki = lax.broadcasted_iota(jnp.int32, (QT, SKV), 1)
            mask = (jnp.abs(qi - ki) <= 128) | (ki < 32) | (qi < 32)
            s = jnp.where(mask, s, -1e9)
            m = jnp.max(s, axis=1, keepdims=True)
            w = jnp.exp(s - m)
            denom = jnp.sum(w, axis=1, keepdims=True)
            p = (w / denom).astype(jnp.bfloat16)
            ctx = lax.dot_general(
                p, v_ref[h], (((1,), (0,)), ((), ())),
                preferred_element_type=jnp.float32)
            ctx_ref[h, 0:QT, :] = ctx.astype(jnp.bfloat16)
            return carry

        lax.fori_loop(0, HP, attn_t0, 0)

        WB = QT + 2 * 128
        for t in range(1, NQT):
            w0 = min(t * QT - 128, SKV - WB)

            def attn_band(h, carry, t=t, w0=w0):
                qh = q_ref[h, t * QT:(t + 1) * QT, :]
                kw = k_ref[h, w0:w0 + WB, :]
                kg = k_ref[h, 0:128, :]
                sb = lax.dot_general(
                    qh, kw, (((1,), (1,)), ((), ())),
                    preferred_element_type=jnp.float32) * SCALE
                sg = lax.dot_general(
                    qh, kg, (((1,), (1,)), ((), ())),
                    preferred_element_type=jnp.float32) * SCALE
                qi = lax.broadcasted_iota(jnp.int32, (QT, WB), 0) + t * QT
                kib = lax.broadcasted_iota(jnp.int32, (QT, WB), 1) + w0
                sb = jnp.where(jnp.abs(qi - kib) <= 128, sb, -1e9)
                kig = lax.broadcasted_iota(jnp.int32, (QT, 128), 1)
                sg = jnp.where(kig < 32, sg, -1e9)
                m = jnp.maximum(jnp.max(sb, axis=1, keepdims=True),
                                jnp.max(sg, axis=1, keepdims=True))
                wb = jnp.exp(sb - m)
                wg = jnp.exp(sg - m)
                denom = (jnp.sum(wb, axis=1, keepdims=True)
                         + jnp.sum(wg, axis=1, keepdims=True))
                ctx = lax.dot_general(
                    (wb / denom).astype(jnp.bfloat16), v_ref[h, w0:w0 + WB, :],
                    (((1,), (0,)), ((), ())),
                    preferred_element_type=jnp.float32)
                ctx = ctx + lax.dot_general(
                    (wg / denom).astype(jnp.bfloat16), v_ref[h, 0:128, :],
                    (((1,), (0,)), ((), ())),
                    preferred_element_type=jnp.float32)
                ctx_ref[h, t * QT:(t + 1) * QT, :] = ctx.astype(jnp.bfloat16)
                return carry

            lax.fori_loop(0, HP, attn_band, 0)

        for rt in range(NQT):
            r0 = rt * QT
            acc = jnp.zeros((QT, DM), jnp.float32)
            for j in range(HP):
                acc = acc + lax.dot_general(
                    ctx_ref[j, r0:r0 + QT, :], wo_ref[j * DH:(j + 1) * DH, :],
                    (((1,), (0,)), ((), ())),
                    preferred_element_type=jnp.float32)
            out_ref[0, r0:r0 + QT, :] = acc

        def rdma(slot, src, dst, dev, sems):
            r = pltpu.make_async_remote_copy(
                src_ref=src, dst_ref=dst,
                send_sem=sems[0].at[slot], recv_sem=sems[1].at[slot],
                device_id=(dev,), device_id_type=pl.DeviceIdType.MESH)
            r.start()
            return r

        for s_ in range(N_DEV - 1):
            cp = lax.rem(my - s_ + N_DEV, N_DEV)
            cm = lax.rem(my + s_, N_DEV)
            ssb_p[...] = out_ref[0, pl.ds(cp * CH, CH), 0:HC].astype(jnp.bfloat16)
            ssb_m[...] = out_ref[0, pl.ds(cm * CH, CH), HC:DM].astype(jnp.bfloat16)
            rp = rdma(s_, ssb_p, rsb_p.at[s_], right, (send_p, recv_p))
            rm = rdma(s_, ssb_m, rsb_m.at[s_], left, (send_m, recv_m))
            rp.wait()
            rm.wait()
            op = lax.rem(my - s_ - 1 + N_DEV, N_DEV) * CH
            om = lax.rem(my + s_ + 1, N_DEV) * CH
            out_ref[0, pl.ds(op, CH), 0:HC] = (
                out_ref[0, pl.ds(op, CH), 0:HC] + rsb_p[s_].astype(jnp.float32))
            out_ref[0, pl.ds(om, CH), HC:DM] = (
                out_ref[0, pl.ds(om, CH), HC:DM] + rsb_m[s_].astype(jnp.float32))

        o_p = lax.rem(my + 1, N_DEV) * CH
        o_m = lax.rem(my - 1 + N_DEV, N_DEV) * CH
        agb[pl.ds(o_p, CH), 0:HC] = out_ref[0, pl.ds(o_p, CH), 0:HC].astype(jnp.bfloat16)
        agb[pl.ds(o_m, CH), HC:DM] = out_ref[0, pl.ds(o_m, CH), HC:DM].astype(jnp.bfloat16)

        for s_ in range(N_DEV - 1):
            slot = N_DEV - 1 + s_
            sp = lax.rem(my + 1 - s_ + N_DEV, N_DEV) * CH
            sm = lax.rem(my - 1 + s_ + N_DEV, N_DEV) * CH
            rp = rdma(slot, agb.at[pl.ds(sp, CH), 0:HC],
                      agb.at[pl.ds(sp, CH), 0:HC], right, (send_p, recv_p))
            rm = rdma(slot, agb.at[pl.ds(sm, CH), HC:DM],
                      agb.at[pl.ds(sm, CH), HC:DM], left, (send_m, recv_m))
            rp.wait()
            rm.wait()
            gp = lax.rem(my - s_ + N_DEV, N_DEV) * CH
            gm = lax.rem(my + s_, N_DEV) * CH
            out_ref[0, pl.ds(gp, CH), 0:HC] = agb[pl.ds(gp, CH), 0:HC].astype(jnp.float32)
            out_ref[0, pl.ds(gm, CH), HC:DM] = agb[pl.ds(gm, CH), HC:DM].astype(jnp.float32)

    kwargs = {}
    if _INTERPRET:
        kwargs["interpret"] = pltpu.InterpretParams()

    return pl.pallas_call(
        body,
        out_shape=jax.ShapeDtypeStruct((1, SQ, DM), jnp.float32),
        in_specs=[
            pl.BlockSpec(memory_space=pltpu.MemorySpace.VMEM),
            pl.BlockSpec(memory_space=pltpu.MemorySpace.VMEM),
            pl.BlockSpec(memory_space=pltpu.MemorySpace.VMEM),
            pl.BlockSpec(memory_space=pltpu.MemorySpace.VMEM),
            pl.BlockSpec(memory_space=pltpu.MemorySpace.VMEM),
        ],
        out_specs=pl.BlockSpec(memory_space=pltpu.MemorySpace.VMEM),
        scratch_shapes=[
            pltpu.VMEM((HP, SQ, DH), jnp.bfloat16),
            pltpu.VMEM((HP, SQ, DH), jnp.bfloat16),
            pltpu.VMEM((SQ, DM), jnp.bfloat16),
            pltpu.VMEM((N_DEV - 1, CH, HC), jnp.bfloat16),
            pltpu.VMEM((N_DEV - 1, CH, HC), jnp.bfloat16),
            pltpu.VMEM((CH, HC), jnp.bfloat16),
            pltpu.VMEM((CH, HC), jnp.bfloat16),
            pltpu.SemaphoreType.DMA((N_HOPS,)),
            pltpu.SemaphoreType.DMA((N_HOPS,)),
            pltpu.SemaphoreType.DMA((N_HOPS,)),
            pltpu.SemaphoreType.DMA((N_HOPS,)),
        ],
        compiler_params=pltpu.CompilerParams(
            collective_id=0,
            vmem_limit_bytes=60 * 1024 * 1024,
        ),
        **kwargs,
    )(xb, wqb, kb, vb, wob)


# device time: 164805 ns/iter; 2.5170x vs baseline; 1.1195x over previous
import os

import jax
import jax.numpy as jnp
from jax import lax
from jax.experimental import pallas as pl
from jax.experimental.pallas import tpu as pltpu

N_DEV = 8
HP = 8
DH = 128
SQ = 2048
SKV = 2048
DM = 1024
HC = DM // 2
CH = SQ // N_DEV
QT = 512
NQT = SQ // QT
SCALE = 0.08838834764831843
N_HOPS = 2 * (N_DEV - 1)

_INTERPRET = os.environ.get("SCBAND_INTERPRET") == "1"


def kernel(x, Wq, K_ext, V_ext, Wo):
    my_out = lax.axis_index("i")
    xb = x.astype(jnp.bfloat16)
    wqb = Wq.astype(jnp.bfloat16)
    wob = Wo.astype(jnp.bfloat16)
    kb = jnp.transpose(
        lax.dynamic_slice(K_ext[0], (0, my_out * HP, 0), (SKV, HP, DH)),
        (1, 0, 2)).astype(jnp.bfloat16)
    vb = jnp.transpose(
        lax.dynamic_slice(V_ext[0], (0, my_out * HP, 0), (SKV, HP, DH)),
        (1, 0, 2)).astype(jnp.bfloat16)

    def body(x_ref, wq_ref, k_ref, v_ref, wo_ref, out_ref,
             q_ref, ctx_ref, acc_bf, rsb_p, rsb_m,
             send_p, recv_p, send_m, recv_m):
        my = lax.axis_index("i")
        p_ = lax.rem(my, 4)
        cz = my // 4
        cy = p_ // 2
        cx = (p_ % 2) ^ cy

        def ring_idx(x, y, z):
            return 4 * z + 2 * y + (x ^ y)

        xp = ring_idx(1 - cx, cy, cz)
        yp = ring_idx(cx, 1 - cy, cz)
        zp = ring_idx(cx, cy, 1 - cz)
        cid = cx + 2 * cy + 4 * cz

        barrier = pltpu.get_barrier_semaphore()
        for nbr in (xp, yp, zp):
            pl.semaphore_signal(barrier, inc=1, device_id=(nbr,),
                                device_id_type=pl.DeviceIdType.MESH)
        pl.semaphore_wait(barrier, 3)

        for rt in range(NQT):
            r0 = rt * QT
            qc = lax.dot_general(
                x_ref[0, r0:r0 + QT, :], wq_ref[...],
                (((1,), (0,)), ((), ())),
                preferred_element_type=jnp.float32).astype(jnp.bfloat16)
            for j in range(HP):
                q_ref[j, r0:r0 + QT, :] = qc[:, j * DH:(j + 1) * DH]

        def attn_t0(h, carry):
            qh = q_ref[h, 0:QT, :]
            s = lax.dot_general(
                qh, k_ref[h], (((1,), (1,)), ((), ())),
                preferred_element_type=jnp.float32) * SCALE
            qi = lax.broadcasted_iota(jnp.int32, (QT, SKV), 0)
            ki = lax.broadcasted_iota(jnp.int32, (QT, SKV), 1)
            mask = (jnp.abs(qi - ki) <= 128) | (ki < 32) | (qi < 32)
            s = jnp.where(mask, s, -1e9)
            m = jnp.max(s, axis=1, keepdims=True)
            w = jnp.exp(s - m)
            denom = jnp.sum(w, axis=1, keepdims=True)
            p = (w / denom).astype(jnp.bfloat16)
            ctx = lax.dot_general(
                p, v_ref[h], (((1,), (0,)), ((), ())),
                preferred_element_type=jnp.float32)
            ctx_ref[h, 0:QT, :] = ctx.astype(jnp.bfloat16)
            return carry

        lax.fori_loop(0, HP, attn_t0, 0)

        WB = QT + 2 * 128
        for t in range(1, NQT):
            w0 = min(t * QT - 128, SKV - WB)

            def attn_band(h, carry, t=t, w0=w0):
                qh = q_ref[h, t * QT:(t + 1) * QT, :]
                kw = k_ref[h, w0:w0 + WB, :]
                kg = k_ref[h, 0:128, :]
                sb = lax.dot_general(
                    qh, kw, (((1,), (1,)), ((), ())),
                    preferred_element_type=jnp.float32) * SCALE
                sg = lax.dot_general(
                    qh, kg, (((1,), (1,)), ((), ())),
                    preferred_element_type=jnp.float32) * SCALE
                qi = lax.broadcasted_iota(jnp.int32, (QT, WB), 0) + t * QT
                kib = lax.broadcasted_iota(jnp.int32, (QT, WB), 1) + w0
                sb = jnp.where(jnp.abs(qi - kib) <= 128, sb, -1e9)
                kig = lax.broadcasted_iota(jnp.int32, (QT, 128), 1)
                sg = jnp.where(kig < 32, sg, -1e9)
                m = jnp.maximum(jnp.max(sb, axis=1, keepdims=True),
                                jnp.max(sg, axis=1, keepdims=True))
                wb = jnp.exp(sb - m)
                wg = jnp.exp(sg - m)
                denom = (jnp.sum(wb, axis=1, keepdims=True)
                         + jnp.sum(wg, axis=1, keepdims=True))
                ctx = lax.dot_general(
                    (wb / denom).astype(jnp.bfloat16), v_ref[h, w0:w0 + WB, :],
                    (((1,), (0,)), ((), ())),
                    preferred_element_type=jnp.float32)
                ctx = ctx + lax.dot_general(
                    (wg / denom).astype(jnp.bfloat16), v_ref[h, 0:128, :],
                    (((1,), (0,)), ((), ())),
                    preferred_element_type=jnp.float32)
                ctx_ref[h, t * QT:(t + 1) * QT, :] = ctx.astype(jnp.bfloat16)
                return carry

            lax.fori_loop(0, HP, attn_band, 0)

        for rt in range(NQT):
            r0 = rt * QT
            acc = jnp.zeros((QT, DM), jnp.float32)
            for j in range(HP):
                acc = acc + lax.dot_general(
                    ctx_ref[j, r0:r0 + QT, :], wo_ref[j * DH:(j + 1) * DH, :],
                    (((1,), (0,)), ((), ())),
                    preferred_element_type=jnp.float32)
            acc_bf[r0:r0 + QT, :] = acc.astype(jnp.bfloat16)

        L, R = (0, HC), (HC, DM)

        def exchange(partner, sends, col, sems, rsb=None, base=0):
            c0, c1 = col
            rs = []
            for i, (sid, rid) in enumerate(sends):
                slot = base + i
                dst = (rsb.at[slot] if rsb is not None
                       else acc_bf.at[pl.ds(sid * CH, CH), c0:c1])
                r = pltpu.make_async_remote_copy(
                    src_ref=acc_bf.at[pl.ds(sid * CH, CH), c0:c1],
                    dst_ref=dst,
                    send_sem=sems[0].at[slot], recv_sem=sems[1].at[slot],
                    device_id=(partner,),
                    device_id_type=pl.DeviceIdType.MESH)
                r.start()
                rs.append(r)
            return rs

        def accumulate(sends, col, rsb, base):
            c0, c1 = col
            for i, (_, rid) in enumerate(sends):
                off = rid * CH
                acc_bf[pl.ds(off, CH), c0:c1] = (
                    acc_bf[pl.ds(off, CH), c0:c1] + rsb[base + i])

        b2_ = [(0, 0), (1, 0), (0, 1), (1, 1)]
        rs_L = [
            (xp, [((1 - cx) + 2 * b1 + 4 * b2, cx + 2 * b1 + 4 * b2)
                  for b1, b2 in b2_]),
            (yp, [(cx + 2 * (1 - cy) + 4 * b2, cx + 2 * cy + 4 * b2)
                  for b2 in (0, 1)]),
            (zp, [(cx + 2 * cy + 4 * (1 - cz), cid)]),
        ]
        rs_R = [
            (yp, [(bx + 2 * (1 - cy) + 4 * bz, bx + 2 * cy + 4 * bz)
                  for bx, bz in b2_]),
            (zp, [(bx + 2 * cy + 4 * (1 - cz), bx + 2 * cy + 4 * cz)
                  for bx in (0, 1)]),
            (xp, [((1 - cx) + 2 * cy + 4 * cz, cid)]),
        ]
        ag_L = [
            (zp, [(cid, 0)]),
            (yp, [(cx + 2 * cy + 4 * b2, 0) for b2 in (0, 1)]),
            (xp, [(cx + 2 * b1 + 4 * b2, 0) for b1, b2 in b2_]),
        ]
        ag_R = [
            (xp, [(cid, 0)]),
            (zp, [(bx + 2 * cy + 4 * cz, 0) for bx in (0, 1)]),
            (yp, [(bx + 2 * cy + 4 * bz, 0) for bx, bz in b2_]),
        ]
        bases = [0, 4, 6]
        ag_bases = [7, 8, 10]

        for st in range(3):
            pl_, sl = rs_L[st]
            pm_, sm_ = rs_R[st]
            rl = exchange(pl_, sl, L, (send_p, recv_p), rsb_p, bases[st])
            rm = exchange(pm_, sm_, R, (send_m, recv_m), rsb_m, bases[st])
            for r in rl + rm:
                r.wait()
            accumulate(sl, L, rsb_p, bases[st])
            accumulate(sm_, R, rsb_m, bases[st])

        for st in range(3):
            pl_, sl = ag_L[st]
            pm_, sm_ = ag_R[st]
            rl = exchange(pl_, sl, L, (send_p, recv_p), None, ag_bases[st])
            rm = exchange(pm_, sm_, R, (send_m, recv_m), None, ag_bases[st])
            for r in rl + rm:
                r.wait()

        out_ref[0] = acc_bf[...].astype(jnp.float32)

    kwargs = {}
    if _INTERPRET:
        kwargs["interpret"] = pltpu.InterpretParams()

    return pl.pallas_call(
        body,
        out_shape=jax.ShapeDtypeStruct((1, SQ, DM), jnp.float32),
        in_specs=[
            pl.BlockSpec(memory_space=pltpu.MemorySpace.VMEM),
            pl.BlockSpec(memory_space=pltpu.MemorySpace.VMEM),
            pl.BlockSpec(memory_space=pltpu.MemorySpace.VMEM),
            pl.BlockSpec(memory_space=pltpu.MemorySpace.VMEM),
            pl.BlockSpec(memory_space=pltpu.MemorySpace.VMEM),
        ],
        out_specs=pl.BlockSpec(memory_space=pltpu.MemorySpace.VMEM),
        scratch_shapes=[
            pltpu.VMEM((HP, SQ, DH), jnp.bfloat16),
            pltpu.VMEM((HP, SQ, DH), jnp.bfloat16),
            pltpu.VMEM((SQ, DM), jnp.bfloat16),
            pltpu.VMEM((N_DEV - 1, CH, HC), jnp.bfloat16),
            pltpu.VMEM((N_DEV - 1, CH, HC), jnp.bfloat16),
            pltpu.SemaphoreType.DMA((N_HOPS,)),
            pltpu.SemaphoreType.DMA((N_HOPS,)),
            pltpu.SemaphoreType.DMA((N_HOPS,)),
            pltpu.SemaphoreType.DMA((N_HOPS,)),
        ],
        compiler_params=pltpu.CompilerParams(
            collective_id=0,
            vmem_limit_bytes=60 * 1024 * 1024,
        ),
        **kwargs,
    )(xb, wqb, kb, vb, wob)


# device time: 159461 ns/iter; 2.6013x vs baseline; 1.0335x over previous
import os

import jax
import jax.numpy as jnp
from jax import lax
from jax.experimental import pallas as pl
from jax.experimental.pallas import tpu as pltpu

N_DEV = 8
HP = 8
DH = 128
SQ = 2048
SKV = 2048
DM = 1024
HC = DM // 2
CH = SQ // N_DEV
QT = 512
NQT = SQ // QT
SCALE = 0.08838834764831843
N_HOPS = 2 * (N_DEV - 1)

_INTERPRET = os.environ.get("SCBAND_INTERPRET") == "1"


def kernel(x, Wq, K_ext, V_ext, Wo):
    my_out = lax.axis_index("i")
    xb = x.astype(jnp.bfloat16)
    wqb = Wq.astype(jnp.bfloat16)
    wob = Wo.astype(jnp.bfloat16)
    kb = jnp.transpose(
        lax.dynamic_slice(K_ext[0], (0, my_out * HP, 0), (SKV, HP, DH)),
        (1, 0, 2)).astype(jnp.bfloat16)
    vb = jnp.transpose(
        lax.dynamic_slice(V_ext[0], (0, my_out * HP, 0), (SKV, HP, DH)),
        (1, 0, 2)).astype(jnp.bfloat16)

    def body(x_ref, wq_ref, k_ref, v_ref, wo_ref, out_ref,
             q_ref, ctx_ref, acc_bf, rsb_p, rsb_m, bias0, biasb, biasg,
             send_p, recv_p, send_m, recv_m):
        my = lax.axis_index("i")
        p_ = lax.rem(my, 4)
        cz = my // 4
        cy = p_ // 2
        cx = (p_ % 2) ^ cy

        def ring_idx(x, y, z):
            return 4 * z + 2 * y + (x ^ y)

        xp = ring_idx(1 - cx, cy, cz)
        yp = ring_idx(cx, 1 - cy, cz)
        zp = ring_idx(cx, cy, 1 - cz)
        cid = cx + 2 * cy + 4 * cz

        barrier = pltpu.get_barrier_semaphore()
        for nbr in (xp, yp, zp):
            pl.semaphore_signal(barrier, inc=1, device_id=(nbr,),
                                device_id_type=pl.DeviceIdType.MESH)
        pl.semaphore_wait(barrier, 3)

        for rt in range(NQT):
            r0 = rt * QT
            qc = lax.dot_general(
                x_ref[0, r0:r0 + QT, :], wq_ref[...],
                (((1,), (0,)), ((), ())),
                preferred_element_type=jnp.float32).astype(jnp.bfloat16)
            for j in range(HP):
                q_ref[j, r0:r0 + QT, :] = qc[:, j * DH:(j + 1) * DH]

        qi = lax.broadcasted_iota(jnp.int32, (QT, SKV), 0)
        ki = lax.broadcasted_iota(jnp.int32, (QT, SKV), 1)
        mask0 = (jnp.abs(qi - ki) <= 128) | (ki < 32) | (qi < 32)
        bias0[...] = jnp.where(mask0, 0.0, -1e9)

        def attn_t0(h, carry):
            qh = q_ref[h, 0:QT, :]
            s = lax.dot_general(
                qh, k_ref[h], (((1,), (1,)), ((), ())),
                preferred_element_type=jnp.float32) * SCALE + bias0[...]
            w = jnp.exp(s)
            r = 1.0 / jnp.sum(w, axis=1, keepdims=True)
            p = (w * r).astype(jnp.bfloat16)
            ctx = lax.dot_general(
                p, v_ref[h], (((1,), (0,)), ((), ())),
                preferred_element_type=jnp.float32)
            ctx_ref[h, 0:QT, :] = ctx.astype(jnp.bfloat16)
            return carry

        lax.fori_loop(0, HP, attn_t0, 0)

        kig = lax.broadcasted_iota(jnp.int32, (QT, 128), 1)
        biasg[...] = jnp.where(kig < 32, 0.0, -1e9)
        WB = QT + 2 * 128
        for t in range(1, NQT):
            w0 = min(t * QT - 128, SKV - WB)
            qib = lax.broadcasted_iota(jnp.int32, (QT, WB), 0) + t * QT
            kib = lax.broadcasted_iota(jnp.int32, (QT, WB), 1) + w0
            biasb[...] = jnp.where(jnp.abs(qib - kib) <= 128, 0.0, -1e9)

            def attn_band(h, carry, t=t, w0=w0):
                qh = q_ref[h, t * QT:(t + 1) * QT, :]
                sb = lax.dot_general(
                    qh, k_ref[h, w0:w0 + WB, :], (((1,), (1,)), ((), ())),
                    preferred_element_type=jnp.float32) * SCALE + biasb[...]
                sg = lax.dot_general(
                    qh, k_ref[h, 0:128, :], (((1,), (1,)), ((), ())),
                    preferred_element_type=jnp.float32) * SCALE + biasg[...]
                wb = jnp.exp(sb)
                wg = jnp.exp(sg)
                r = 1.0 / (jnp.sum(wb, axis=1, keepdims=True)
                           + jnp.sum(wg, axis=1, keepdims=True))
                ctx = lax.dot_general(
                    (wb * r).astype(jnp.bfloat16), v_ref[h, w0:w0 + WB, :],
                    (((1,), (0,)), ((), ())),
                    preferred_element_type=jnp.float32)
                ctx = ctx + lax.dot_general(
                    (wg * r).astype(jnp.bfloat16), v_ref[h, 0:128, :],
                    (((1,), (0,)), ((), ())),
                    preferred_element_type=jnp.float32)
                ctx_ref[h, t * QT:(t + 1) * QT, :] = ctx.astype(jnp.bfloat16)
                return carry

            lax.fori_loop(0, HP, attn_band, 0)

        for rt in range(NQT):
            r0 = rt * QT
            acc = jnp.zeros((QT, DM), jnp.float32)
            for j in range(HP):
                acc = acc + lax.dot_general(
                    ctx_ref[j, r0:r0 + QT, :], wo_ref[j * DH:(j + 1) * DH, :],
                    (((1,), (0,)), ((), ())),
                    preferred_element_type=jnp.float32)
            acc_bf[r0:r0 + QT, :] = acc.astype(jnp.bfloat16)

        L, R = (0, HC), (HC, DM)

        def exchange(partner, sends, col, sems, rsb=None, base=0):
            c0, c1 = col
            rs = []
            for i, (sid, rid) in enumerate(sends):
                slot = base + i
                dst = (rsb.at[slot] if rsb is not None
                       else acc_bf.at[pl.ds(sid * CH, CH), c0:c1])
                r = pltpu.make_async_remote_copy(
                    src_ref=acc_bf.at[pl.ds(sid * CH, CH), c0:c1],
                    dst_ref=dst,
                    send_sem=sems[0].at[slot], recv_sem=sems[1].at[slot],
                    device_id=(partner,),
                    device_id_type=pl.DeviceIdType.MESH)
                r.start()
                rs.append(r)
            return rs

        def accumulate(sends, col, rsb, base):
            c0, c1 = col
            for i, (_, rid) in enumerate(sends):
                off = rid * CH
                acc_bf[pl.ds(off, CH), c0:c1] = (
                    acc_bf[pl.ds(off, CH), c0:c1] + rsb[base + i])

        b2_ = [(0, 0), (1, 0), (0, 1), (1, 1)]
        rs_L = [
            (xp, [((1 - cx) + 2 * b1 + 4 * b2, cx + 2 * b1 + 4 * b2)
                  for b1, b2 in b2_]),
            (yp, [(cx + 2 * (1 - cy) + 4 * b2, cx + 2 * cy + 4 * b2)
                  for b2 in (0, 1)]),
            (zp, [(cx + 2 * cy + 4 * (1 - cz), cid)]),
        ]
        rs_R = [
            (yp, [(bx + 2 * (1 - cy) + 4 * bz, bx + 2 * cy + 4 * bz)
                  for bx, bz in b2_]),
            (zp, [(bx + 2 * cy + 4 * (1 - cz), bx + 2 * cy + 4 * cz)
                  for bx in (0, 1)]),
            (xp, [((1 - cx) + 2 * cy + 4 * cz, cid)]),
        ]
        ag_L = [
            (zp, [(cid, 0)]),
            (yp, [(cx + 2 * cy + 4 * b2, 0) for b2 in (0, 1)]),
            (xp, [(cx + 2 * b1 + 4 * b2, 0) for b1, b2 in b2_]),
        ]
        ag_R = [
            (xp, [(cid, 0)]),
            (zp, [(bx + 2 * cy + 4 * cz, 0) for bx in (0, 1)]),
            (yp, [(bx + 2 * cy + 4 * bz, 0) for bx, bz in b2_]),
        ]
        bases = [0, 4, 6]
        ag_bases = [7, 8, 10]

        for st in range(3):
            pl_, sl = rs_L[st]
            pm_, sm_ = rs_R[st]
            rl = exchange(pl_, sl, L, (send_p, recv_p), rsb_p, bases[st])
            rm = exchange(pm_, sm_, R, (send_m, recv_m), rsb_m, bases[st])
            for r in rl + rm:
                r.wait()
            accumulate(sl, L, rsb_p, bases[st])
            accumulate(sm_, R, rsb_m, bases[st])

        for st in range(3):
            pl_, sl = ag_L[st]
            pm_, sm_ = ag_R[st]
            rl = exchange(pl_, sl, L, (send_p, recv_p), None, ag_bases[st])
            rm = exchange(pm_, sm_, R, (send_m, recv_m), None, ag_bases[st])
            for r in rl + rm:
                r.wait()

        out_ref[0] = acc_bf[...].astype(jnp.float32)

    kwargs = {}
    if _INTERPRET:
        kwargs["interpret"] = pltpu.InterpretParams()

    return pl.pallas_call(
        body,
        out_shape=jax.ShapeDtypeStruct((1, SQ, DM), jnp.float32),
        in_specs=[
            pl.BlockSpec(memory_space=pltpu.MemorySpace.VMEM),
            pl.BlockSpec(memory_space=pltpu.MemorySpace.VMEM),
            pl.BlockSpec(memory_space=pltpu.MemorySpace.VMEM),
            pl.BlockSpec(memory_space=pltpu.MemorySpace.VMEM),
            pl.BlockSpec(memory_space=pltpu.MemorySpace.VMEM),
        ],
        out_specs=pl.BlockSpec(memory_space=pltpu.MemorySpace.VMEM),
        scratch_shapes=[
            pltpu.VMEM((HP, SQ, DH), jnp.bfloat16),
            pltpu.VMEM((HP, SQ, DH), jnp.bfloat16),
            pltpu.VMEM((SQ, DM), jnp.bfloat16),
            pltpu.VMEM((N_DEV - 1, CH, HC), jnp.bfloat16),
            pltpu.VMEM((N_DEV - 1, CH, HC), jnp.bfloat16),
            pltpu.VMEM((QT, SKV), jnp.float32),
            pltpu.VMEM((QT, QT + 256), jnp.float32),
            pltpu.VMEM((QT, 128), jnp.float32),
            pltpu.SemaphoreType.DMA((N_HOPS,)),
            pltpu.SemaphoreType.DMA((N_HOPS,)),
            pltpu.SemaphoreType.DMA((N_HOPS,)),
            pltpu.SemaphoreType.DMA((N_HOPS,)),
        ],
        compiler_params=pltpu.CompilerParams(
            collective_id=0,
            vmem_limit_bytes=60 * 1024 * 1024,
        ),
        **kwargs,
    )(xb, wqb, kb, vb, wob)


# device time: 137929 ns/iter; 3.0074x vs baseline; 1.1561x over previous
import os

import jax
import jax.numpy as jnp
from jax import lax
from jax.experimental import pallas as pl
from jax.experimental.pallas import tpu as pltpu

N_DEV = 8
HP = 8
DH = 128
SQ = 2048
SKV = 2048
DM = 1024
HC = DM // 2
CH = SQ // N_DEV
QT = 512
NQT = SQ // QT
SCALE = 0.08838834764831843
N_HOPS = 2 * (N_DEV - 1)

_INTERPRET = os.environ.get("SCBAND_INTERPRET") == "1"


def kernel(x, Wq, K_ext, V_ext, Wo):
    my_out = lax.axis_index("i")
    xb = x.astype(jnp.bfloat16)
    wqb = Wq.astype(jnp.bfloat16)
    wob = Wo.astype(jnp.bfloat16)
    kb = lax.dynamic_slice(
        K_ext[0], (0, my_out * HP, 0), (SKV, HP, DH)
    ).astype(jnp.bfloat16).reshape(SKV, HP * DH)
    vb = lax.dynamic_slice(
        V_ext[0], (0, my_out * HP, 0), (SKV, HP, DH)
    ).astype(jnp.bfloat16).reshape(SKV, HP * DH)

    def body(x_ref, wq_ref, k_ref, v_ref, wo_ref, out_ref,
             q_ref, ctx_ref, acc_bf, rsb_p, rsb_m, bias0, biasb, biasg,
             send_p, recv_p, send_m, recv_m):
        my = lax.axis_index("i")
        p_ = lax.rem(my, 4)
        cz = my // 4
        cy = p_ // 2
        cx = (p_ % 2) ^ cy

        def ring_idx(x, y, z):
            return 4 * z + 2 * y + (x ^ y)

        xp = ring_idx(1 - cx, cy, cz)
        yp = ring_idx(cx, 1 - cy, cz)
        zp = ring_idx(cx, cy, 1 - cz)
        cid = cx + 2 * cy + 4 * cz

        barrier = pltpu.get_barrier_semaphore()
        for nbr in (xp, yp, zp):
            pl.semaphore_signal(barrier, inc=1, device_id=(nbr,),
                                device_id_type=pl.DeviceIdType.MESH)
        pl.semaphore_wait(barrier, 3)

        for rt in range(NQT):
            r0 = rt * QT
            q_ref[r0:r0 + QT, :] = lax.dot_general(
                x_ref[0, r0:r0 + QT, :], wq_ref[...],
                (((1,), (0,)), ((), ())),
                preferred_element_type=jnp.float32).astype(jnp.bfloat16)

        qi = lax.broadcasted_iota(jnp.int32, (QT, SKV), 0)
        ki = lax.broadcasted_iota(jnp.int32, (QT, SKV), 1)
        mask0 = (jnp.abs(qi - ki) <= 128) | (ki < 32) | (qi < 32)
        bias0[...] = jnp.where(mask0, 0.0, -1e9)

        for j in range(HP):
            c0, c1 = j * DH, (j + 1) * DH
            s = lax.dot_general(
                q_ref[0:QT, c0:c1], k_ref[:, c0:c1], (((1,), (1,)), ((), ())),
                preferred_element_type=jnp.float32) * SCALE + bias0[...]
            w = jnp.exp(s)
            r = 1.0 / jnp.sum(w, axis=1, keepdims=True)
            p = (w * r).astype(jnp.bfloat16)
            ctx_ref[0:QT, c0:c1] = lax.dot_general(
                p, v_ref[:, c0:c1], (((1,), (0,)), ((), ())),
                preferred_element_type=jnp.float32).astype(jnp.bfloat16)

        kig = lax.broadcasted_iota(jnp.int32, (QT, 128), 1)
        biasg[...] = jnp.where(kig < 32, 0.0, -1e9)
        WB = QT + 2 * 128
        for t in range(1, NQT):
            w0 = min(t * QT - 128, SKV - WB)
            qib = lax.broadcasted_iota(jnp.int32, (QT, WB), 0) + t * QT
            kib = lax.broadcasted_iota(jnp.int32, (QT, WB), 1) + w0
            biasb[...] = jnp.where(jnp.abs(qib - kib) <= 128, 0.0, -1e9)

            for j in range(HP):
                c0, c1 = j * DH, (j + 1) * DH
                qh = q_ref[t * QT:(t + 1) * QT, c0:c1]
                sb = lax.dot_general(
                    qh, k_ref[w0:w0 + WB, c0:c1], (((1,), (1,)), ((), ())),
                    preferred_element_type=jnp.float32) * SCALE + biasb[...]
                sg = lax.dot_general(
                    qh, k_ref[0:128, c0:c1], (((1,), (1,)), ((), ())),
                    preferred_element_type=jnp.float32) * SCALE + biasg[...]
                wb = jnp.exp(sb)
                wg = jnp.exp(sg)
                r = 1.0 / (jnp.sum(wb, axis=1, keepdims=True)
                           + jnp.sum(wg, axis=1, keepdims=True))
                ctx = lax.dot_general(
                    (wb * r).astype(jnp.bfloat16), v_ref[w0:w0 + WB, c0:c1],
                    (((1,), (0,)), ((), ())),
                    preferred_element_type=jnp.float32)
                ctx = ctx + lax.dot_general(
                    (wg * r).astype(jnp.bfloat16), v_ref[0:128, c0:c1],
                    (((1,), (0,)), ((), ())),
                    preferred_element_type=jnp.float32)
                ctx_ref[t * QT:(t + 1) * QT, c0:c1] = ctx.astype(jnp.bfloat16)

        for rt in range(NQT):
            r0 = rt * QT
            acc_bf[r0:r0 + QT, :] = lax.dot_general(
                ctx_ref[r0:r0 + QT, :], wo_ref[...],
                (((1,), (0,)), ((), ())),
                preferred_element_type=jnp.float32).astype(jnp.bfloat16)

        L, R = (0, HC), (HC, DM)

        def exchange(partner, sends, col, sems, rsb=None, base=0):
            c0, c1 = col
            rs = []
            for i, (sid, rid) in enumerate(sends):
                slot = base + i
                dst = (rsb.at[slot] if rsb is not None
                       else acc_bf.at[pl.ds(sid * CH, CH), c0:c1])
                r = pltpu.make_async_remote_copy(
                    src_ref=acc_bf.at[pl.ds(sid * CH, CH), c0:c1],
                    dst_ref=dst,
                    send_sem=sems[0].at[slot], recv_sem=sems[1].at[slot],
                    device_id=(partner,),
                    device_id_type=pl.DeviceIdType.MESH)
                r.start()
                rs.append(r)
            return rs

        def accumulate(sends, col, rsb, base):
            c0, c1 = col
            for i, (_, rid) in enumerate(sends):
                off = rid * CH
                acc_bf[pl.ds(off, CH), c0:c1] = (
                    acc_bf[pl.ds(off, CH), c0:c1] + rsb[base + i])

        b2_ = [(0, 0), (1, 0), (0, 1), (1, 1)]
        rs_L = [
            (xp, [((1 - cx) + 2 * b1 + 4 * b2, cx + 2 * b1 + 4 * b2)
                  for b1, b2 in b2_]),
            (yp, [(cx + 2 * (1 - cy) + 4 * b2, cx + 2 * cy + 4 * b2)
                  for b2 in (0, 1)]),
            (zp, [(cx + 2 * cy + 4 * (1 - cz), cid)]),
        ]
        rs_R = [
            (yp, [(bx + 2 * (1 - cy) + 4 * bz, bx + 2 * cy + 4 * bz)
                  for bx, bz in b2_]),
            (zp, [(bx + 2 * cy + 4 * (1 - cz), bx + 2 * cy + 4 * cz)
                  for bx in (0, 1)]),
            (xp, [((1 - cx) + 2 * cy + 4 * cz, cid)]),
        ]
        ag_L = [
            (zp, [(cid, 0)]),
            (yp, [(cx + 2 * cy + 4 * b2, 0) for b2 in (0, 1)]),
            (xp, [(cx + 2 * b1 + 4 * b2, 0) for b1, b2 in b2_]),
        ]
        ag_R = [
            (xp, [(cid, 0)]),
            (zp, [(bx + 2 * cy + 4 * cz, 0) for bx in (0, 1)]),
            (yp, [(bx + 2 * cy + 4 * bz, 0) for bx, bz in b2_]),
        ]
        bases = [0, 4, 6]
        ag_bases = [7, 8, 10]

        for st in range(3):
            pl_, sl = rs_L[st]
            pm_, sm_ = rs_R[st]
            rl = exchange(pl_, sl, L, (send_p, recv_p), rsb_p, bases[st])
            rm = exchange(pm_, sm_, R, (send_m, recv_m), rsb_m, bases[st])
            for r in rl + rm:
                r.wait()
            accumulate(sl, L, rsb_p, bases[st])
            accumulate(sm_, R, rsb_m, bases[st])

        for st in range(3):
            pl_, sl = ag_L[st]
            pm_, sm_ = ag_R[st]
            rl = exchange(pl_, sl, L, (send_p, recv_p), None, ag_bases[st])
            rm = exchange(pm_, sm_, R, (send_m, recv_m), None, ag_bases[st])
            for r in rl + rm:
                r.wait()

        out_ref[0] = acc_bf[...].astype(jnp.float32)

    kwargs = {}
    if _INTERPRET:
        kwargs["interpret"] = pltpu.InterpretParams()

    return pl.pallas_call(
        body,
        out_shape=jax.ShapeDtypeStruct((1, SQ, DM), jnp.float32),
        in_specs=[
            pl.BlockSpec(memory_space=pltpu.MemorySpace.VMEM),
            pl.BlockSpec(memory_space=pltpu.MemorySpace.VMEM),
            pl.BlockSpec(memory_space=pltpu.MemorySpace.VMEM),
            pl.BlockSpec(memory_space=pltpu.MemorySpace.VMEM),
            pl.BlockSpec(memory_space=pltpu.MemorySpace.VMEM),
        ],
        out_specs=pl.BlockSpec(memory_space=pltpu.MemorySpace.VMEM),
        scratch_shapes=[
            pltpu.VMEM((SQ, DM), jnp.bfloat16),
            pltpu.VMEM((SQ, DM), jnp.bfloat16),
            pltpu.VMEM((SQ, DM), jnp.bfloat16),
            pltpu.VMEM((N_DEV - 1, CH, HC), jnp.bfloat16),
            pltpu.VMEM((N_DEV - 1, CH, HC), jnp.bfloat16),
            pltpu.VMEM((QT, SKV), jnp.float32),
            pltpu.VMEM((QT, QT + 256), jnp.float32),
            pltpu.VMEM((QT, 128), jnp.float32),
            pltpu.SemaphoreType.DMA((N_HOPS,)),
            pltpu.SemaphoreType.DMA((N_HOPS,)),
            pltpu.SemaphoreType.DMA((N_HOPS,)),
            pltpu.SemaphoreType.DMA((N_HOPS,)),
        ],
        compiler_params=pltpu.CompilerParams(
            collective_id=0,
            vmem_limit_bytes=60 * 1024 * 1024,
        ),
        **kwargs,
    )(xb, wqb, kb, vb, wob)
